# baseline (speedup 1.0000x reference)
import functools

import jax
import jax.numpy as jnp
from jax import lax
from jax.experimental import pallas as pl
from jax.experimental.pallas import tpu as pltpu
from jax.experimental.pallas import tpu_sc as plsc

N_EXPERTS = 8
LANES = 16
N_WORKERS = 32
TOKEN_BLOCK = 1024


def _logits_body(x_ref, w_ref, out_ref):
    out_ref[...] = lax.dot_general(
        w_ref[...], x_ref[...],
        dimension_numbers=(((1,), (1,)), ((), ())),
        preferred_element_type=jnp.float32,
    )


def _compute_logits_t(x, W):
    n_tokens, d = x.shape
    grid = (n_tokens // TOKEN_BLOCK,)
    return pl.pallas_call(
        _logits_body,
        grid=grid,
        in_specs=[
            pl.BlockSpec((TOKEN_BLOCK, d), lambda i: (i, 0)),
            pl.BlockSpec((N_EXPERTS, d), lambda i: (0, 0)),
        ],
        out_specs=pl.BlockSpec((N_EXPERTS, TOKEN_BLOCK), lambda i: (0, i)),
        out_shape=jax.ShapeDtypeStruct((N_EXPERTS, n_tokens), jnp.float32),
    )(x, W)


def _make_router(n_tokens):
    tpw = n_tokens // N_WORKERS
    mesh = plsc.VectorSubcoreMesh(core_axis_name="c", subcore_axis_name="s")

    @functools.partial(
        pl.kernel,
        out_type=[
            jax.ShapeDtypeStruct((2, n_tokens), jnp.float32),
            jax.ShapeDtypeStruct((2, n_tokens), jnp.int32),
        ],
        mesh=mesh,
        scratch_types=[
            pltpu.VMEM((N_EXPERTS, tpw), jnp.float32),
            pltpu.VMEM((2, tpw), jnp.float32),
            pltpu.VMEM((2, tpw), jnp.int32),
        ],
    )
    def route(lt_hbm, ow_hbm, oi_hbm, lbuf, wbuf, ibuf):
        wid = lax.axis_index("s") * 2 + lax.axis_index("c")
        base = wid * tpw
        pltpu.sync_copy(lt_hbm.at[:, pl.ds(base, tpw)], lbuf)

        def one_group(off):
            m1 = lbuf[0, pl.ds(off, LANES)]
            i1 = jnp.zeros((LANES,), jnp.int32)
            m2 = jnp.full((LANES,), -3e38, jnp.float32)
            i2 = jnp.zeros((LANES,), jnp.int32)
            for e in range(1, N_EXPERTS):
                v = lbuf[e, pl.ds(off, LANES)]
                gt1 = v > m1
                gt2 = v > m2
                ev = jnp.full((LANES,), e, jnp.int32)
                i2 = jnp.where(gt1, i1, jnp.where(gt2, ev, i2))
                m2 = jnp.where(gt1, m1, jnp.where(gt2, v, m2))
                i1 = jnp.where(gt1, ev, i1)
                m1 = jnp.where(gt1, v, m1)
            dexp = jnp.exp(m2 - m1)
            w1 = 1.0 / (1.0 + dexp)
            wbuf[0, pl.ds(off, LANES)] = w1
            wbuf[1, pl.ds(off, LANES)] = dexp * w1
            ibuf[0, pl.ds(off, LANES)] = i1
            ibuf[1, pl.ds(off, LANES)] = i2
        def group(g, carry):
            one_group(g * LANES)
            return carry

        lax.fori_loop(0, tpw // LANES, group, 0)
        pltpu.sync_copy(wbuf, ow_hbm.at[:, pl.ds(base, tpw)])
        pltpu.sync_copy(ibuf, oi_hbm.at[:, pl.ds(base, tpw)])

    return route


def kernel(x, W):
    n_tokens = x.shape[0]
    logits_t = _compute_logits_t(x, W)
    wt, it = _make_router(n_tokens)(logits_t)
    return wt.T, it.T


# speedup vs baseline: 1.1659x; 1.1659x over previous
import functools

import jax
import jax.numpy as jnp
from jax import lax
from jax.experimental import pallas as pl
from jax.experimental.pallas import tpu as pltpu
from jax.experimental.pallas import tpu_sc as plsc

N_EXPERTS = 8
LANES = 16
N_WORKERS = 32
TOKEN_BLOCK = 4096


def _logits_body(x_ref, w_ref, out_ref):
    out_ref[...] = lax.dot_general(
        w_ref[...], x_ref[...],
        dimension_numbers=(((1,), (1,)), ((), ())),
        preferred_element_type=jnp.float32,
    )


def _compute_logits_t(x, W):
    n_tokens, d = x.shape
    grid = (n_tokens // TOKEN_BLOCK,)
    return pl.pallas_call(
        _logits_body,
        grid=grid,
        in_specs=[
            pl.BlockSpec((TOKEN_BLOCK, d), lambda i: (i, 0)),
            pl.BlockSpec((N_EXPERTS, d), lambda i: (0, 0)),
        ],
        out_specs=pl.BlockSpec((N_EXPERTS, TOKEN_BLOCK), lambda i: (0, i)),
        out_shape=jax.ShapeDtypeStruct((N_EXPERTS, n_tokens), jnp.float32),
    )(x, W)


def _make_router(n_tokens):
    tpw = n_tokens // N_WORKERS
    mesh = plsc.VectorSubcoreMesh(core_axis_name="c", subcore_axis_name="s")

    @functools.partial(
        pl.kernel,
        out_type=[
            jax.ShapeDtypeStruct((2, n_tokens), jnp.float32),
            jax.ShapeDtypeStruct((2, n_tokens), jnp.int32),
        ],
        mesh=mesh,
        scratch_types=[
            pltpu.VMEM((N_EXPERTS, tpw), jnp.float32),
            pltpu.VMEM((2, tpw), jnp.float32),
            pltpu.VMEM((2, tpw), jnp.int32),
        ],
    )
    def route(lt_hbm, ow_hbm, oi_hbm, lbuf, wbuf, ibuf):
        wid = lax.axis_index("s") * 2 + lax.axis_index("c")
        base = wid * tpw
        pltpu.sync_copy(lt_hbm.at[:, pl.ds(base, tpw)], lbuf)

        def one_group(off):
            m1 = lbuf[0, pl.ds(off, LANES)]
            i1 = jnp.zeros((LANES,), jnp.int32)
            m2 = jnp.full((LANES,), -3e38, jnp.float32)
            i2 = jnp.zeros((LANES,), jnp.int32)
            for e in range(1, N_EXPERTS):
                v = lbuf[e, pl.ds(off, LANES)]
                gt1 = v > m1
                gt2 = v > m2
                ev = jnp.full((LANES,), e, jnp.int32)
                i2 = jnp.where(gt1, i1, jnp.where(gt2, ev, i2))
                m2 = jnp.where(gt1, m1, jnp.where(gt2, v, m2))
                i1 = jnp.where(gt1, ev, i1)
                m1 = jnp.where(gt1, v, m1)
            dexp = jnp.exp(m2 - m1)
            w1 = 1.0 / (1.0 + dexp)
            wbuf[0, pl.ds(off, LANES)] = w1
            wbuf[1, pl.ds(off, LANES)] = dexp * w1
            ibuf[0, pl.ds(off, LANES)] = i1
            ibuf[1, pl.ds(off, LANES)] = i2
        def group(g, carry):
            one_group(g * LANES)
            return carry

        lax.fori_loop(0, tpw // LANES, group, 0)
        pltpu.sync_copy(wbuf, ow_hbm.at[:, pl.ds(base, tpw)])
        pltpu.sync_copy(ibuf, oi_hbm.at[:, pl.ds(base, tpw)])

    return route


def kernel(x, W):
    n_tokens = x.shape[0]
    logits_t = _compute_logits_t(x, W)
    wt, it = _make_router(n_tokens)(logits_t)
    return wt.T, it.T


# speedup vs baseline: 1.1794x; 1.0116x over previous
import functools

import jax
import jax.numpy as jnp
from jax import lax
from jax.experimental import pallas as pl
from jax.experimental.pallas import tpu as pltpu
from jax.experimental.pallas import tpu_sc as plsc

N_EXPERTS = 8
LANES = 16
N_WORKERS = 32
TOKEN_BLOCK = 4096


def _logits_body(x_ref, w_ref, out_ref):
    acc = lax.dot_general(
        w_ref[...], x_ref[...],
        dimension_numbers=(((1,), (1,)), ((), ())),
        preferred_element_type=jnp.float32,
    )
    i = lax.bitcast_convert_type(acc, jnp.int32)
    enc = i ^ ((i >> 31) & jnp.int32(0x7FFFFFFF))
    eid = (N_EXPERTS - 1) - lax.broadcasted_iota(jnp.int32, acc.shape, 0)
    out_ref[...] = (enc & jnp.int32(~7)) | eid


def _compute_logits_t(x, W):
    n_tokens, d = x.shape
    grid = (n_tokens // TOKEN_BLOCK,)
    return pl.pallas_call(
        _logits_body,
        grid=grid,
        in_specs=[
            pl.BlockSpec((TOKEN_BLOCK, d), lambda i: (i, 0)),
            pl.BlockSpec((N_EXPERTS, d), lambda i: (0, 0)),
        ],
        out_specs=pl.BlockSpec((N_EXPERTS, TOKEN_BLOCK), lambda i: (0, i)),
        out_shape=jax.ShapeDtypeStruct((N_EXPERTS, n_tokens), jnp.int32),
    )(x, W)


def _make_router(n_tokens):
    tpw = n_tokens // N_WORKERS
    mesh = plsc.VectorSubcoreMesh(core_axis_name="c", subcore_axis_name="s")

    @functools.partial(
        pl.kernel,
        out_type=[
            jax.ShapeDtypeStruct((2, n_tokens), jnp.float32),
            jax.ShapeDtypeStruct((2, n_tokens), jnp.int32),
        ],
        mesh=mesh,
        scratch_types=[
            pltpu.VMEM((N_EXPERTS, tpw), jnp.int32),
            pltpu.VMEM((2, tpw), jnp.float32),
            pltpu.VMEM((2, tpw), jnp.int32),
        ],
    )
    def route(lt_hbm, ow_hbm, oi_hbm, lbuf, wbuf, ibuf):
        wid = lax.axis_index("s") * 2 + lax.axis_index("c")
        base = wid * tpw
        pltpu.sync_copy(lt_hbm.at[:, pl.ds(base, tpw)], lbuf)

        def one_group(off):
            m1 = lbuf[0, pl.ds(off, LANES)]
            m2 = jnp.full((LANES,), jnp.iinfo(jnp.int32).min, jnp.int32)
            for e in range(1, N_EXPERTS):
                k = lbuf[e, pl.ds(off, LANES)]
                m2 = jnp.maximum(m2, jnp.minimum(m1, k))
                m1 = jnp.maximum(m1, k)
            seven = jnp.full((LANES,), 7, jnp.int32)
            sgn = jnp.full((LANES,), 0x7FFFFFFF, jnp.int32)
            l1 = lax.bitcast_convert_type(m1 ^ ((m1 >> 31) & sgn), jnp.float32)
            l2 = lax.bitcast_convert_type(m2 ^ ((m2 >> 31) & sgn), jnp.float32)
            dexp = jnp.exp(l2 - l1)
            w1 = 1.0 / (1.0 + dexp)
            wbuf[0, pl.ds(off, LANES)] = w1
            wbuf[1, pl.ds(off, LANES)] = dexp * w1
            ibuf[0, pl.ds(off, LANES)] = seven - (m1 & seven)
            ibuf[1, pl.ds(off, LANES)] = seven - (m2 & seven)
        def group(g, carry):
            one_group(g * LANES)
            return carry

        lax.fori_loop(0, tpw // LANES, group, 0)
        pltpu.sync_copy(wbuf, ow_hbm.at[:, pl.ds(base, tpw)])
        pltpu.sync_copy(ibuf, oi_hbm.at[:, pl.ds(base, tpw)])

    return route


def kernel(x, W):
    n_tokens = x.shape[0]
    logits_t = _compute_logits_t(x, W)
    wt, it = _make_router(n_tokens)(logits_t)
    return wt.T, it.T
